# baseline (speedup 1.0000x reference)
import functools

import jax
import jax.numpy as jnp
from jax import lax
from jax.experimental import pallas as pl
from jax.experimental.pallas import tpu as pltpu
from jax.experimental.pallas import tpu_sc as plsc

N = 100000
EMB = 128
ATOM_V = 93
RES_V = 30
NCHAIN = 2
EPS = 1e-06

PREP_BLOCK = 25000
PREP_STEPS = N // PREP_BLOCK
T2_ROWS = ATOM_V * RES_V * NCHAIN

SC_NC = 2
SC_NS = 16
SC_NW = SC_NC * SC_NS
SUB = 80
NTASK = N // SUB
TASKS_PER_W = -(-NTASK // SC_NW)


def _prep_body(z_ref, r_ref, c_ref, at_ref, rt_ref, idx_ref, t2_ref,
               gz_ref, gr_ref):
    i = pl.program_id(0)

    z = z_ref[0, 0, :]
    r = r_ref[0, 0, :]
    c = c_ref[0, 0, :]

    idx_ref[0, 0, :] = (z * (RES_V * NCHAIN) + r * NCHAIN) + c

    @pl.when(i == 0)
    def _init():
        gz_ref[...] = jnp.zeros_like(gz_ref)
        gr_ref[...] = jnp.zeros_like(gr_ref)

    oz = (z[:, None] == lax.broadcasted_iota(jnp.int32, (PREP_BLOCK, ATOM_V), 1)
          ).astype(jnp.bfloat16)
    orr = (r[:, None] == lax.broadcasted_iota(jnp.int32, (PREP_BLOCK, RES_V), 1)
           ).astype(jnp.bfloat16)
    oc = (c[:, None] == lax.broadcasted_iota(jnp.int32, (PREP_BLOCK, NCHAIN), 1)
          ).astype(jnp.bfloat16)

    gz_ref[...] += lax.dot_general(oc, oz, (((0,), (0,)), ((), ())),
                                   preferred_element_type=jnp.float32)
    gr_ref[...] += lax.dot_general(oc, orr, (((0,), (0,)), ((), ())),
                                   preferred_element_type=jnp.float32)

    @pl.when(i == PREP_STEPS - 1)
    def _finish():
        gz = gz_ref[...]
        gr = gr_ref[...]
        s = (jnp.dot(gz, at_ref[...], preferred_element_type=jnp.float32)
             + jnp.dot(gr, rt_ref[...], preferred_element_type=jnp.float32))
        cnt = jnp.sum(gz, axis=1)
        mean = s / (cnt[:, None] + EPS)
        left = (at_ref[...][:, None, None, :]
                + rt_ref[...][None, :, None, :])
        left = jnp.broadcast_to(left, (ATOM_V, RES_V, NCHAIN, EMB))
        right = jnp.broadcast_to(mean[None, None, :, :],
                                 (ATOM_V, RES_V, NCHAIN, EMB))
        t2 = jnp.concatenate([left, right], axis=-1)
        t2_ref[...] = t2.reshape(T2_ROWS, 2 * EMB)


def _prep(z3, r3, c3, atom_table, residue_table):
    grid = (PREP_STEPS,)
    iblock = pl.BlockSpec((1, 1, PREP_BLOCK), lambda i: (i, 0, 0))
    full_at = pl.BlockSpec((ATOM_V, EMB), lambda i: (0, 0))
    full_rt = pl.BlockSpec((RES_V, EMB), lambda i: (0, 0))
    return pl.pallas_call(
        _prep_body,
        grid=grid,
        in_specs=[iblock, iblock, iblock, full_at, full_rt],
        out_specs=[iblock, pl.BlockSpec((T2_ROWS, 2 * EMB), lambda i: (0, 0))],
        out_shape=[
            jax.ShapeDtypeStruct((PREP_STEPS, 1, PREP_BLOCK), jnp.int32),
            jax.ShapeDtypeStruct((T2_ROWS, 2 * EMB), jnp.float32),
        ],
        scratch_shapes=[
            pltpu.VMEM((NCHAIN, ATOM_V), jnp.float32),
            pltpu.VMEM((NCHAIN, RES_V), jnp.float32),
        ],
    )(z3, r3, c3, atom_table, residue_table)


NBUF = 4
LOOK = 2
EXTRA = NTASK % SC_NW
BASE_TASKS = NTASK // SC_NW


def _sc_body(t2_hbm, idx_hbm, out_hbm, idx_all, row_bufs, isem, gsems, wsems):
    w = lax.axis_index("s") * SC_NC + lax.axis_index("c")
    start = w * BASE_TASKS + jnp.minimum(w, EXTRA)
    n_w = BASE_TASKS + (w < EXTRA).astype(jnp.int32)
    row0 = start * SUB

    nbase = BASE_TASKS * SUB
    pltpu.async_copy(idx_hbm.at[pl.ds(row0, nbase)],
                     idx_all.at[pl.ds(0, nbase)], isem).wait()

    @pl.when(w < EXTRA)
    def _extra():
        pltpu.async_copy(idx_hbm.at[pl.ds(row0 + nbase, SUB)],
                         idx_all.at[pl.ds(nbase, SUB)], isem).wait()

    def fire_gather(k, b):
        @pl.when(k < n_w)
        def _do():
            pltpu.async_copy(t2_hbm.at[idx_all.at[pl.ds(k * SUB, SUB)]],
                             row_bufs.at[b], gsems.at[b])

    def wait_gather_fire_write(k, b):
        @pl.when(k < n_w)
        def _do():
            pltpu.make_async_copy(t2_hbm.at[idx_all.at[pl.ds(k * SUB, SUB)]],
                                  row_bufs.at[b], gsems.at[b]).wait()
            pltpu.async_copy(
                row_bufs.at[b],
                out_hbm.at[pl.ds(row0 + k * SUB, SUB)], wsems.at[b])

    def wait_write(k, b):
        @pl.when(jnp.logical_and(k >= 0, k < n_w))
        def _do():
            pltpu.make_async_copy(
                row_bufs.at[b],
                out_hbm.at[pl.ds(row0 + k * SUB, SUB)], wsems.at[b]).wait()

    for j in range(LOOK):
        fire_gather(jnp.int32(j), j % NBUF)

    n_turns = -(-TASKS_PER_W // NBUF)

    def turn(g, _):
        for b in range(NBUF):
            k = g * NBUF + b
            wait_gather_fire_write(k, b)
            wait_write(k + LOOK - NBUF, (b + LOOK) % NBUF)
            fire_gather(k + LOOK, (b + LOOK) % NBUF)
        return 0

    lax.fori_loop(0, n_turns, turn, 0)

    last = n_turns * NBUF
    for j in range(last - (NBUF - LOOK), last):
        wait_write(jnp.int32(j), j % NBUF)


@functools.cache
def _sc_gather():
    return pl.kernel(
        _sc_body,
        out_type=jax.ShapeDtypeStruct((N, 2 * EMB), jnp.float32),
        mesh=plsc.VectorSubcoreMesh(core_axis_name="c", subcore_axis_name="s",
                                    num_cores=SC_NC, num_subcores=SC_NS),
        scratch_types=[
            pltpu.VMEM((TASKS_PER_W * SUB,), jnp.int32),
            pltpu.VMEM((NBUF, SUB, 2 * EMB), jnp.float32),
            pltpu.SemaphoreType.DMA,
            pltpu.SemaphoreType.DMA((NBUF,)),
            pltpu.SemaphoreType.DMA((NBUF,)),
        ],
    )


def kernel(Z, residue_types, chain_ids, atom_table, residue_table):
    z = Z.astype(jnp.int32)
    r = residue_types.astype(jnp.int32)
    c = chain_ids.astype(jnp.int32)
    idx = (z * (RES_V * NCHAIN) + r * NCHAIN) + c
    h = jnp.take(atom_table, z, axis=0) + jnp.take(residue_table, r, axis=0)
    sum_chain = jnp.zeros((NCHAIN, EMB), jnp.float32).at[c].add(h)
    cnt = jnp.zeros((NCHAIN,), jnp.float32).at[c].add(1.0)
    mean = sum_chain / (cnt[:, None] + EPS)
    left = jnp.broadcast_to(
        atom_table[:, None, None, :] + residue_table[None, :, None, :],
        (ATOM_V, RES_V, NCHAIN, EMB))
    right = jnp.broadcast_to(mean[None, None, :, :],
                             (ATOM_V, RES_V, NCHAIN, EMB))
    t2 = jnp.concatenate([left, right], axis=-1).reshape(T2_ROWS, 2 * EMB)
    return _sc_gather()(t2, idx)


# speedup vs baseline: 7.3761x; 7.3761x over previous
import functools

import jax
import jax.numpy as jnp
from jax import lax
from jax.experimental import pallas as pl
from jax.experimental.pallas import tpu as pltpu
from jax.experimental.pallas import tpu_sc as plsc

N = 100000
EMB = 128
ATOM_V = 93
RES_V = 30
NCHAIN = 2
EPS = 1e-06

PREP_BLOCK = 25000
PREP_STEPS = N // PREP_BLOCK
T2_ROWS = ATOM_V * RES_V * NCHAIN

SC_NC = 2
SC_NS = 16
SC_NW = SC_NC * SC_NS
SUB = 80
NTASK = N // SUB
TASKS_PER_W = -(-NTASK // SC_NW)


def _prep_body(zrc_ref, at_ref, rt_ref, idx_ref, t2_ref, g_ref):
    i = pl.program_id(0)

    z = zrc_ref[0, 0, 0, :]
    r = zrc_ref[1, 0, 0, :]
    c = zrc_ref[2, 0, 0, :]

    idx_ref[0, 0, :] = (z * (RES_V * NCHAIN) + r * NCHAIN) + c

    @pl.when(i == 0)
    def _init():
        g_ref[...] = jnp.zeros_like(g_ref)

    iota = lax.broadcasted_iota(jnp.int32, (PREP_BLOCK, 128), 1)
    oh = ((z[:, None] == iota) | ((r[:, None] + ATOM_V) == iota)
          ).astype(jnp.bfloat16)
    w2 = jnp.concatenate(
        [jnp.ones((1, PREP_BLOCK), jnp.bfloat16),
         c.astype(jnp.bfloat16)[None, :]], axis=0)
    g_ref[...] += jnp.dot(w2, oh, preferred_element_type=jnp.float32)

    @pl.when(i == PREP_STEPS - 1)
    def _finish():
        g = g_ref[...]
        tcomb = jnp.concatenate(
            [at_ref[...], rt_ref[...],
             jnp.zeros((128 - ATOM_V - RES_V, EMB), jnp.float32)], axis=0)
        s = jnp.dot(g, tcomb, preferred_element_type=jnp.float32)
        cnt1 = jnp.sum(g[1, :]) * 0.5
        cnt0 = float(N) - cnt1
        mean1 = s[1:2, :] / (cnt1 + EPS)
        mean0 = (s[0:1, :] - s[1:2, :]) / (cnt0 + EPS)
        mean = jnp.concatenate([mean0, mean1], axis=0)
        left = (at_ref[...][:, None, None, :]
                + rt_ref[...][None, :, None, :])
        left = jnp.broadcast_to(left, (ATOM_V, RES_V, NCHAIN, EMB))
        right = jnp.broadcast_to(mean[None, None, :, :],
                                 (ATOM_V, RES_V, NCHAIN, EMB))
        t2 = jnp.concatenate([left, right], axis=-1)
        t2_ref[...] = t2.reshape(T2_ROWS, 2 * EMB)


def _prep(zrc4, atom_table, residue_table):
    grid = (PREP_STEPS,)
    zblock = pl.BlockSpec((3, 1, 1, PREP_BLOCK), lambda i: (0, i, 0, 0))
    iblock = pl.BlockSpec((1, 1, PREP_BLOCK), lambda i: (i, 0, 0))
    full_at = pl.BlockSpec((ATOM_V, EMB), lambda i: (0, 0))
    full_rt = pl.BlockSpec((RES_V, EMB), lambda i: (0, 0))
    return pl.pallas_call(
        _prep_body,
        grid=grid,
        in_specs=[zblock, full_at, full_rt],
        out_specs=[iblock, pl.BlockSpec((T2_ROWS, 2 * EMB), lambda i: (0, 0))],
        out_shape=[
            jax.ShapeDtypeStruct((PREP_STEPS, 1, PREP_BLOCK), jnp.int32),
            jax.ShapeDtypeStruct((T2_ROWS, 2 * EMB), jnp.float32),
        ],
        scratch_shapes=[
            pltpu.VMEM((NCHAIN, 128), jnp.float32),
        ],
    )(zrc4, atom_table, residue_table)


NBUF = 4
LOOK = 2
EXTRA = NTASK % SC_NW
BASE_TASKS = NTASK // SC_NW


def _sc_body(t2_hbm, idx_hbm, out_hbm, idx_all, row_bufs, isem, gsems, wsems):
    w = lax.axis_index("s") * SC_NC + lax.axis_index("c")
    start = w * BASE_TASKS + jnp.minimum(w, EXTRA)
    n_w = BASE_TASKS + (w < EXTRA).astype(jnp.int32)
    row0 = start * SUB

    nbase = BASE_TASKS * SUB
    pltpu.async_copy(idx_hbm.at[pl.ds(row0, nbase)],
                     idx_all.at[pl.ds(0, nbase)], isem).wait()

    @pl.when(w < EXTRA)
    def _extra():
        pltpu.async_copy(idx_hbm.at[pl.ds(row0 + nbase, SUB)],
                         idx_all.at[pl.ds(nbase, SUB)], isem).wait()

    def fire_gather(k, b):
        @pl.when(k < n_w)
        def _do():
            pltpu.async_copy(t2_hbm.at[idx_all.at[pl.ds(k * SUB, SUB)]],
                             row_bufs.at[b], gsems.at[b])

    def wait_gather_fire_write(k, b):
        @pl.when(k < n_w)
        def _do():
            pltpu.make_async_copy(t2_hbm.at[idx_all.at[pl.ds(k * SUB, SUB)]],
                                  row_bufs.at[b], gsems.at[b]).wait()
            pltpu.async_copy(
                row_bufs.at[b],
                out_hbm.at[pl.ds(row0 + k * SUB, SUB)], wsems.at[b])

    def wait_write(k, b):
        @pl.when(jnp.logical_and(k >= 0, k < n_w))
        def _do():
            pltpu.make_async_copy(
                row_bufs.at[b],
                out_hbm.at[pl.ds(row0 + k * SUB, SUB)], wsems.at[b]).wait()

    for j in range(LOOK):
        fire_gather(jnp.int32(j), j % NBUF)

    n_turns = -(-TASKS_PER_W // NBUF)

    def turn(g, _):
        for b in range(NBUF):
            k = g * NBUF + b
            wait_gather_fire_write(k, b)
            wait_write(k + LOOK - NBUF, (b + LOOK) % NBUF)
            fire_gather(k + LOOK, (b + LOOK) % NBUF)
        return 0

    lax.fori_loop(0, n_turns, turn, 0)

    last = n_turns * NBUF
    for j in range(last - (NBUF - LOOK), last):
        wait_write(jnp.int32(j), j % NBUF)


@functools.cache
def _sc_gather():
    return pl.kernel(
        _sc_body,
        out_type=jax.ShapeDtypeStruct((N, 2 * EMB), jnp.float32),
        mesh=plsc.VectorSubcoreMesh(core_axis_name="c", subcore_axis_name="s",
                                    num_cores=SC_NC, num_subcores=SC_NS),
        scratch_types=[
            pltpu.VMEM((TASKS_PER_W * SUB,), jnp.int32),
            pltpu.VMEM((NBUF, SUB, 2 * EMB), jnp.float32),
            pltpu.SemaphoreType.DMA,
            pltpu.SemaphoreType.DMA((NBUF,)),
            pltpu.SemaphoreType.DMA((NBUF,)),
        ],
    )


def kernel(Z, residue_types, chain_ids, atom_table, residue_table):
    zrc = jnp.stack([Z, residue_types, chain_ids]).astype(jnp.int32)
    zrc4 = zrc.reshape(3, PREP_STEPS, 1, PREP_BLOCK)
    idx3, t2 = _prep(zrc4, atom_table, residue_table)
    idx = idx3.reshape(N)
    return _sc_gather()(t2, idx)


# speedup vs baseline: 7.4011x; 1.0034x over previous
import functools

import jax
import jax.numpy as jnp
from jax import lax
from jax.experimental import pallas as pl
from jax.experimental.pallas import tpu as pltpu
from jax.experimental.pallas import tpu_sc as plsc

N = 100000
EMB = 128
ATOM_V = 93
RES_V = 30
NCHAIN = 2
EPS = 1e-06

PREP_BLOCK = 50000
PREP_STEPS = N // PREP_BLOCK
T2_ROWS = ATOM_V * RES_V * NCHAIN

SC_NC = 2
SC_NS = 16
SC_NW = SC_NC * SC_NS
SUB = 80
NTASK = N // SUB
TASKS_PER_W = -(-NTASK // SC_NW)


def _prep_body(zrc_ref, at_ref, rt_ref, idx_ref, t2_ref, g_ref):
    i = pl.program_id(0)

    z = zrc_ref[0, 0, 0, :]
    r = zrc_ref[1, 0, 0, :]
    c = zrc_ref[2, 0, 0, :]

    idx_ref[0, 0, :] = (z * (RES_V * NCHAIN) + r * NCHAIN) + c

    @pl.when(i == 0)
    def _init():
        g_ref[...] = jnp.zeros_like(g_ref)

    iota = lax.broadcasted_iota(jnp.int32, (PREP_BLOCK, 128), 1)
    oh = ((z[:, None] == iota) | ((r[:, None] + ATOM_V) == iota)
          ).astype(jnp.bfloat16)
    w2 = jnp.concatenate(
        [jnp.ones((1, PREP_BLOCK), jnp.bfloat16),
         c.astype(jnp.bfloat16)[None, :]], axis=0)
    g_ref[...] += jnp.dot(w2, oh, preferred_element_type=jnp.float32)

    @pl.when(i == PREP_STEPS - 1)
    def _finish():
        g = g_ref[...]
        tcomb = jnp.concatenate(
            [at_ref[...], rt_ref[...],
             jnp.zeros((128 - ATOM_V - RES_V, EMB), jnp.float32)], axis=0)
        s = jnp.dot(g, tcomb, preferred_element_type=jnp.float32)
        cnt1 = jnp.sum(g[1, :]) * 0.5
        cnt0 = float(N) - cnt1
        mean1 = s[1:2, :] / (cnt1 + EPS)
        mean0 = (s[0:1, :] - s[1:2, :]) / (cnt0 + EPS)
        mean = jnp.concatenate([mean0, mean1], axis=0)
        left = (at_ref[...][:, None, None, :]
                + rt_ref[...][None, :, None, :])
        left = jnp.broadcast_to(left, (ATOM_V, RES_V, NCHAIN, EMB))
        right = jnp.broadcast_to(mean[None, None, :, :],
                                 (ATOM_V, RES_V, NCHAIN, EMB))
        t2 = jnp.concatenate([left, right], axis=-1)
        t2_ref[...] = t2.reshape(T2_ROWS, 2 * EMB)


def _prep(zrc4, atom_table, residue_table):
    grid = (PREP_STEPS,)
    zblock = pl.BlockSpec((3, 1, 1, PREP_BLOCK), lambda i: (0, i, 0, 0))
    iblock = pl.BlockSpec((1, 1, PREP_BLOCK), lambda i: (i, 0, 0))
    full_at = pl.BlockSpec((ATOM_V, EMB), lambda i: (0, 0))
    full_rt = pl.BlockSpec((RES_V, EMB), lambda i: (0, 0))
    return pl.pallas_call(
        _prep_body,
        grid=grid,
        in_specs=[zblock, full_at, full_rt],
        out_specs=[iblock, pl.BlockSpec((T2_ROWS, 2 * EMB), lambda i: (0, 0))],
        out_shape=[
            jax.ShapeDtypeStruct((PREP_STEPS, 1, PREP_BLOCK), jnp.int32),
            jax.ShapeDtypeStruct((T2_ROWS, 2 * EMB), jnp.float32),
        ],
        scratch_shapes=[
            pltpu.VMEM((NCHAIN, 128), jnp.float32),
        ],
    )(zrc4, atom_table, residue_table)


NBUF = 4
LOOK = 2
EXTRA = NTASK % SC_NW
BASE_TASKS = NTASK // SC_NW


def _sc_body(t2_hbm, idx_hbm, out_hbm, idx_all, row_bufs, isem, gsems, wsems):
    w = lax.axis_index("s") * SC_NC + lax.axis_index("c")
    start = w * BASE_TASKS + jnp.minimum(w, EXTRA)
    n_w = BASE_TASKS + (w < EXTRA).astype(jnp.int32)
    row0 = start * SUB

    nbase = BASE_TASKS * SUB
    pltpu.async_copy(idx_hbm.at[pl.ds(row0, nbase)],
                     idx_all.at[pl.ds(0, nbase)], isem).wait()

    @pl.when(w < EXTRA)
    def _extra():
        pltpu.async_copy(idx_hbm.at[pl.ds(row0 + nbase, SUB)],
                         idx_all.at[pl.ds(nbase, SUB)], isem).wait()

    def fire_gather(k, b):
        @pl.when(k < n_w)
        def _do():
            pltpu.async_copy(t2_hbm.at[idx_all.at[pl.ds(k * SUB, SUB)]],
                             row_bufs.at[b], gsems.at[b])

    def wait_gather_fire_write(k, b):
        @pl.when(k < n_w)
        def _do():
            pltpu.make_async_copy(t2_hbm.at[idx_all.at[pl.ds(k * SUB, SUB)]],
                                  row_bufs.at[b], gsems.at[b]).wait()
            pltpu.async_copy(
                row_bufs.at[b],
                out_hbm.at[pl.ds(row0 + k * SUB, SUB)], wsems.at[b])

    def wait_write(k, b):
        @pl.when(jnp.logical_and(k >= 0, k < n_w))
        def _do():
            pltpu.make_async_copy(
                row_bufs.at[b],
                out_hbm.at[pl.ds(row0 + k * SUB, SUB)], wsems.at[b]).wait()

    for j in range(LOOK):
        fire_gather(jnp.int32(j), j % NBUF)

    n_turns = -(-TASKS_PER_W // NBUF)

    def turn(g, _):
        for b in range(NBUF):
            k = g * NBUF + b
            wait_gather_fire_write(k, b)
            wait_write(k + LOOK - NBUF, (b + LOOK) % NBUF)
            fire_gather(k + LOOK, (b + LOOK) % NBUF)
        return 0

    lax.fori_loop(0, n_turns, turn, 0)

    last = n_turns * NBUF
    for j in range(last - (NBUF - LOOK), last):
        wait_write(jnp.int32(j), j % NBUF)


@functools.cache
def _sc_gather():
    return pl.kernel(
        _sc_body,
        out_type=jax.ShapeDtypeStruct((N, 2 * EMB), jnp.float32),
        mesh=plsc.VectorSubcoreMesh(core_axis_name="c", subcore_axis_name="s",
                                    num_cores=SC_NC, num_subcores=SC_NS),
        scratch_types=[
            pltpu.VMEM((TASKS_PER_W * SUB,), jnp.int32),
            pltpu.VMEM((NBUF, SUB, 2 * EMB), jnp.float32),
            pltpu.SemaphoreType.DMA,
            pltpu.SemaphoreType.DMA((NBUF,)),
            pltpu.SemaphoreType.DMA((NBUF,)),
        ],
    )


def kernel(Z, residue_types, chain_ids, atom_table, residue_table):
    zrc = jnp.stack([Z, residue_types, chain_ids]).astype(jnp.int32)
    zrc4 = zrc.reshape(3, PREP_STEPS, 1, PREP_BLOCK)
    idx3, t2 = _prep(zrc4, atom_table, residue_table)
    idx = idx3.reshape(N)
    return _sc_gather()(t2, idx)


# speedup vs baseline: 7.4400x; 1.0053x over previous
import functools

import jax
import jax.numpy as jnp
from jax import lax
from jax.experimental import pallas as pl
from jax.experimental.pallas import tpu as pltpu
from jax.experimental.pallas import tpu_sc as plsc

N = 100000
EMB = 128
ATOM_V = 93
RES_V = 30
NCHAIN = 2
EPS = 1e-06

PREP_BLOCK = 50000
PREP_STEPS = N // PREP_BLOCK
T2_ROWS = ATOM_V * RES_V * NCHAIN

SC_NC = 2
SC_NS = 16
SC_NW = SC_NC * SC_NS
SUB = 80
NTASK = N // SUB
TASKS_PER_W = -(-NTASK // SC_NW)


def _prep_body(zrc_ref, at_ref, rt_ref, idx_ref, t2_ref, g_ref):
    i = pl.program_id(0)

    z = zrc_ref[0, 0, 0, :]
    r = zrc_ref[1, 0, 0, :]
    c = zrc_ref[2, 0, 0, :]

    idx_ref[0, 0, :] = (z * (RES_V * NCHAIN) + r * NCHAIN) + c

    @pl.when(i == 0)
    def _init():
        g_ref[...] = jnp.zeros_like(g_ref)

    iota = lax.broadcasted_iota(jnp.int32, (PREP_BLOCK, 128), 1)
    oh = ((z[:, None] == iota) | ((r[:, None] + ATOM_V) == iota)
          ).astype(jnp.bfloat16)
    w2 = jnp.concatenate(
        [jnp.ones((1, PREP_BLOCK), jnp.bfloat16),
         c.astype(jnp.bfloat16)[None, :]], axis=0)
    g_ref[...] += jnp.dot(w2, oh, preferred_element_type=jnp.float32)

    @pl.when(i == PREP_STEPS - 1)
    def _finish():
        g = g_ref[...]
        tcomb = jnp.concatenate(
            [at_ref[...], rt_ref[...],
             jnp.zeros((128 - ATOM_V - RES_V, EMB), jnp.float32)], axis=0)
        s = jnp.dot(g, tcomb, preferred_element_type=jnp.float32)
        cnt1 = jnp.sum(g[1, :]) * 0.5
        cnt0 = float(N) - cnt1
        mean1 = s[1:2, :] / (cnt1 + EPS)
        mean0 = (s[0:1, :] - s[1:2, :]) / (cnt0 + EPS)
        mean = jnp.concatenate([mean0, mean1], axis=0)
        left = (at_ref[...][:, None, None, :]
                + rt_ref[...][None, :, None, :])
        left = jnp.broadcast_to(left, (ATOM_V, RES_V, NCHAIN, EMB))
        right = jnp.broadcast_to(mean[None, None, :, :],
                                 (ATOM_V, RES_V, NCHAIN, EMB))
        t2 = jnp.concatenate([left, right], axis=-1)
        t2_ref[...] = t2.reshape(T2_ROWS, 2 * EMB)


def _prep(zrc4, atom_table, residue_table):
    grid = (PREP_STEPS,)
    zblock = pl.BlockSpec((3, 1, 1, PREP_BLOCK), lambda i: (0, i, 0, 0))
    iblock = pl.BlockSpec((1, 1, PREP_BLOCK), lambda i: (i, 0, 0))
    full_at = pl.BlockSpec((ATOM_V, EMB), lambda i: (0, 0))
    full_rt = pl.BlockSpec((RES_V, EMB), lambda i: (0, 0))
    return pl.pallas_call(
        _prep_body,
        grid=grid,
        in_specs=[zblock, full_at, full_rt],
        out_specs=[iblock, pl.BlockSpec((T2_ROWS, 2 * EMB), lambda i: (0, 0))],
        out_shape=[
            jax.ShapeDtypeStruct((PREP_STEPS, 1, PREP_BLOCK), jnp.int32),
            jax.ShapeDtypeStruct((T2_ROWS, 2 * EMB), jnp.float32),
        ],
        scratch_shapes=[
            pltpu.VMEM((NCHAIN, 128), jnp.float32),
        ],
    )(zrc4, atom_table, residue_table)


NBUF = 5
LOOK = 3
EXTRA = NTASK % SC_NW
BASE_TASKS = NTASK // SC_NW


def _sc_body(t2_hbm, idx_hbm, out_hbm, idx_all, row_bufs, isem, gsems, wsems):
    w = lax.axis_index("s") * SC_NC + lax.axis_index("c")
    start = w * BASE_TASKS + jnp.minimum(w, EXTRA)
    n_w = BASE_TASKS + (w < EXTRA).astype(jnp.int32)
    row0 = start * SUB

    nbase = BASE_TASKS * SUB
    pltpu.async_copy(idx_hbm.at[pl.ds(row0, nbase)],
                     idx_all.at[pl.ds(0, nbase)], isem).wait()

    @pl.when(w < EXTRA)
    def _extra():
        pltpu.async_copy(idx_hbm.at[pl.ds(row0 + nbase, SUB)],
                         idx_all.at[pl.ds(nbase, SUB)], isem).wait()

    def fire_gather(k, b):
        @pl.when(k < n_w)
        def _do():
            pltpu.async_copy(t2_hbm.at[idx_all.at[pl.ds(k * SUB, SUB)]],
                             row_bufs.at[b], gsems.at[b])

    def wait_gather_fire_write(k, b):
        @pl.when(k < n_w)
        def _do():
            pltpu.make_async_copy(t2_hbm.at[idx_all.at[pl.ds(k * SUB, SUB)]],
                                  row_bufs.at[b], gsems.at[b]).wait()
            pltpu.async_copy(
                row_bufs.at[b],
                out_hbm.at[pl.ds(row0 + k * SUB, SUB)], wsems.at[b])

    def wait_write(k, b):
        @pl.when(jnp.logical_and(k >= 0, k < n_w))
        def _do():
            pltpu.make_async_copy(
                row_bufs.at[b],
                out_hbm.at[pl.ds(row0 + k * SUB, SUB)], wsems.at[b]).wait()

    for j in range(LOOK):
        fire_gather(jnp.int32(j), j % NBUF)

    n_turns = -(-TASKS_PER_W // NBUF)

    def turn(g, _):
        for b in range(NBUF):
            k = g * NBUF + b
            wait_gather_fire_write(k, b)
            wait_write(k + LOOK - NBUF, (b + LOOK) % NBUF)
            fire_gather(k + LOOK, (b + LOOK) % NBUF)
        return 0

    lax.fori_loop(0, n_turns, turn, 0)

    last = n_turns * NBUF
    for j in range(last - (NBUF - LOOK), last):
        wait_write(jnp.int32(j), j % NBUF)


@functools.cache
def _sc_gather():
    return pl.kernel(
        _sc_body,
        out_type=jax.ShapeDtypeStruct((N, 2 * EMB), jnp.float32),
        mesh=plsc.VectorSubcoreMesh(core_axis_name="c", subcore_axis_name="s",
                                    num_cores=SC_NC, num_subcores=SC_NS),
        scratch_types=[
            pltpu.VMEM((TASKS_PER_W * SUB,), jnp.int32),
            pltpu.VMEM((NBUF, SUB, 2 * EMB), jnp.float32),
            pltpu.SemaphoreType.DMA,
            pltpu.SemaphoreType.DMA((NBUF,)),
            pltpu.SemaphoreType.DMA((NBUF,)),
        ],
    )


def kernel(Z, residue_types, chain_ids, atom_table, residue_table):
    zrc = jnp.stack([Z, residue_types, chain_ids]).astype(jnp.int32)
    zrc4 = zrc.reshape(3, PREP_STEPS, 1, PREP_BLOCK)
    idx3, t2 = _prep(zrc4, atom_table, residue_table)
    idx = idx3.reshape(N)
    return _sc_gather()(t2, idx)
